# Initial kernel scaffold; baseline (speedup 1.0000x reference)
import functools

import jax
import jax.numpy as jnp
from jax import lax
from jax.experimental import pallas as pl
from jax.experimental.pallas import tpu as pltpu
from jax.experimental.pallas import tpu_sc as plsc

N = 10000
E = 320000
D = 128
G = 64
NC = 2
NS = 16
EK = 80
ROWS_PER_SUB = N // NS
EDGES_PER_SUB = E // (NC * NS)
NCHUNK = EDGES_PER_SUB // EK
NB = 10
BN = N // NB


def _sc_mesh():
    return plsc.VectorSubcoreMesh(core_axis_name="c", subcore_axis_name="s")


def _deg_body(dst_hbm, zeros_hbm, ones_hbm, out_hbm, didx_v, ones_v, sem,
              deg_sh):
    c = lax.axis_index("c")
    s = lax.axis_index("s")
    r0 = s * ROWS_PER_SUB
    pltpu.sync_copy(zeros_hbm.at[pl.ds(r0, ROWS_PER_SUB)],
                    deg_sh.at[pl.ds(r0, ROWS_PER_SUB)])
    pltpu.sync_copy(ones_hbm, ones_v)
    plsc.subcore_barrier()
    base = c * (E // NC) + s * EDGES_PER_SUB

    def step(i, carry):
        off = base + i * EK
        pltpu.sync_copy(dst_hbm.at[pl.ds(off, EK)], didx_v)
        pltpu.sync_copy(ones_v, deg_sh.at[didx_v], add=True)
        return carry

    lax.fori_loop(0, NCHUNK, step, 0)
    plsc.subcore_barrier()
    pltpu.sync_copy(deg_sh.at[pl.ds(r0, ROWS_PER_SUB)],
                    out_hbm.at[c, pl.ds(r0, ROWS_PER_SUB)])


def _sc_degree(dst, zeros_nk, ones_ek):
    return pl.kernel(
        _deg_body,
        out_type=jax.ShapeDtypeStruct((NC, N, 16), jnp.float32),
        mesh=_sc_mesh(),
        scratch_types=[
            pltpu.VMEM((EK,), jnp.int32),
            pltpu.VMEM((EK, 16), jnp.float32),
            pltpu.SemaphoreType.DMA,
            pltpu.VMEM_SHARED((N, 16), jnp.float32),
        ],
    )(dst, zeros_nk, ones_ek)


def _scatter_body(g_hbm, src_hbm, dst_hbm, out_hbm, sidx_v, didx_v, rows_v,
                  sem, acc_sh):
    c = lax.axis_index("c")
    s = lax.axis_index("s")
    r0 = s * ROWS_PER_SUB
    pltpu.sync_copy(g_hbm.at[pl.ds(r0, ROWS_PER_SUB)],
                    acc_sh.at[pl.ds(r0, ROWS_PER_SUB)])
    plsc.subcore_barrier()
    base = c * (E // NC) + s * EDGES_PER_SUB

    def step(i, carry):
        off = base + i * EK
        pltpu.sync_copy(src_hbm.at[pl.ds(off, EK)], sidx_v)
        pltpu.sync_copy(dst_hbm.at[pl.ds(off, EK)], didx_v)
        pltpu.async_copy(g_hbm.at[sidx_v], rows_v, sem).wait()
        pltpu.sync_copy(rows_v, acc_sh.at[didx_v], add=True)
        return carry

    lax.fori_loop(0, NCHUNK, step, 0)
    plsc.subcore_barrier()
    pltpu.sync_copy(acc_sh.at[pl.ds(r0, ROWS_PER_SUB)],
                    out_hbm.at[c, pl.ds(r0, ROWS_PER_SUB)])


def _sc_scatter(g, src, dst):
    return pl.kernel(
        _scatter_body,
        out_type=jax.ShapeDtypeStruct((NC, N, D), jnp.float32),
        mesh=_sc_mesh(),
        scratch_types=[
            pltpu.VMEM((EK,), jnp.int32),
            pltpu.VMEM((EK,), jnp.int32),
            pltpu.VMEM((EK, D), jnp.float32),
            pltpu.SemaphoreType.DMA,
            pltpu.VMEM_SHARED((N, D), jnp.float32),
        ],
    )(g, src, dst)


def _tc1_body(x_ref, w_ref, deg_ref, g_ref):
    deg = deg_ref[0, :, 0:1] + deg_ref[1, :, 0:1] + 1.0
    dinv = lax.rsqrt(deg)
    h = jnp.dot(x_ref[...], w_ref[...], preferred_element_type=jnp.float32)
    g_ref[...] = h * dinv


def _tc_stage1(x, W1, deg2):
    return pl.pallas_call(
        _tc1_body,
        grid=(NB,),
        in_specs=[
            pl.BlockSpec((BN, D), lambda i: (i, 0)),
            pl.BlockSpec((D, D), lambda i: (0, 0)),
            pl.BlockSpec((NC, BN, 16), lambda i: (0, i, 0)),
        ],
        out_specs=pl.BlockSpec((BN, D), lambda i: (i, 0)),
        out_shape=jax.ShapeDtypeStruct((N, D), jnp.float32),
    )(x, W1, deg2)


def _tc2_body(acc_ref, g1_ref, deg_ref, w2_ref, b1_ref, batch_ref,
              g2_ref, p1_ref, p1_acc):
    i = pl.program_id(0)
    deg = deg_ref[0, :, 0:1] + deg_ref[1, :, 0:1] + 1.0
    dinv = lax.rsqrt(deg)
    esum = acc_ref[0] + acc_ref[1] - g1_ref[...]
    out1 = jax.nn.relu(esum * dinv + b1_ref[...])
    h2 = jnp.dot(out1, w2_ref[...], preferred_element_type=jnp.float32)
    g2_ref[...] = h2 * dinv
    onehot = (batch_ref[0] == lax.broadcasted_iota(jnp.int32, (G, BN), 0)
              ).astype(jnp.float32)
    part = jnp.dot(onehot, out1, preferred_element_type=jnp.float32)

    @pl.when(i == 0)
    def _():
        p1_acc[...] = jnp.zeros_like(p1_acc)

    p1_acc[...] += part

    @pl.when(i == NB - 1)
    def _():
        p1_ref[...] = p1_acc[...]


def _tc_stage2(acc1, g1, deg2, W2, b1r, batch3):
    return pl.pallas_call(
        _tc2_body,
        grid=(NB,),
        in_specs=[
            pl.BlockSpec((NC, BN, D), lambda i: (0, i, 0)),
            pl.BlockSpec((BN, D), lambda i: (i, 0)),
            pl.BlockSpec((NC, BN, 16), lambda i: (0, i, 0)),
            pl.BlockSpec((D, D), lambda i: (0, 0)),
            pl.BlockSpec((1, D), lambda i: (0, 0)),
            pl.BlockSpec((1, 1, BN), lambda i: (i, 0, 0)),
        ],
        out_specs=[
            pl.BlockSpec((BN, D), lambda i: (i, 0)),
            pl.BlockSpec((G, D), lambda i: (0, 0)),
        ],
        out_shape=[
            jax.ShapeDtypeStruct((N, D), jnp.float32),
            jax.ShapeDtypeStruct((G, D), jnp.float32),
        ],
        scratch_shapes=[pltpu.VMEM((G, D), jnp.float32)],
    )(acc1, g1, deg2, W2, b1r, batch3)


def _tc3_body(acc_ref, g2_ref, deg_ref, b2_ref, batch_ref, p1_ref,
              wl1_ref, bl1_ref, wl2_ref, bl2_ref, h_ref, lsm_ref, p2_acc):
    i = pl.program_id(0)
    deg = deg_ref[0, :, 0:1] + deg_ref[1, :, 0:1] + 1.0
    dinv = lax.rsqrt(deg)
    esum = acc_ref[0] + acc_ref[1] - g2_ref[...]
    out2 = jax.nn.relu(esum * dinv + b2_ref[...])
    onehot = (batch_ref[0] == lax.broadcasted_iota(jnp.int32, (G, BN), 0)
              ).astype(jnp.float32)
    part = jnp.dot(onehot, out2, preferred_element_type=jnp.float32)

    @pl.when(i == 0)
    def _():
        p2_acc[...] = jnp.zeros_like(p2_acc)

    p2_acc[...] += part

    @pl.when(i == NB - 1)
    def _():
        p = jnp.concatenate([p1_ref[...], p2_acc[...]], axis=1)
        h = jnp.dot(p, wl1_ref[...], preferred_element_type=jnp.float32)
        h = jax.nn.relu(h + bl1_ref[...])
        h = jnp.dot(h, wl2_ref[...], preferred_element_type=jnp.float32)
        h = h + bl2_ref[...]
        m = jnp.max(h, axis=1, keepdims=True)
        lse = jnp.log(jnp.sum(jnp.exp(h - m), axis=1, keepdims=True))
        h_ref[...] = h
        lsm_ref[...] = h - m - lse


def _tc_stage3(acc2, g2, deg2, b2r, batch3, p1, Wl1, bl1r, Wl2, bl2r):
    return pl.pallas_call(
        _tc3_body,
        grid=(NB,),
        in_specs=[
            pl.BlockSpec((NC, BN, D), lambda i: (0, i, 0)),
            pl.BlockSpec((BN, D), lambda i: (i, 0)),
            pl.BlockSpec((NC, BN, 16), lambda i: (0, i, 0)),
            pl.BlockSpec((1, D), lambda i: (0, 0)),
            pl.BlockSpec((1, 1, BN), lambda i: (i, 0, 0)),
            pl.BlockSpec((G, D), lambda i: (0, 0)),
            pl.BlockSpec((2 * D, 2 * D), lambda i: (0, 0)),
            pl.BlockSpec((1, 2 * D), lambda i: (0, 0)),
            pl.BlockSpec((2 * D, 10), lambda i: (0, 0)),
            pl.BlockSpec((1, 10), lambda i: (0, 0)),
        ],
        out_specs=[
            pl.BlockSpec((G, 10), lambda i: (0, 0)),
            pl.BlockSpec((G, 10), lambda i: (0, 0)),
        ],
        out_shape=[
            jax.ShapeDtypeStruct((G, 10), jnp.float32),
            jax.ShapeDtypeStruct((G, 10), jnp.float32),
        ],
        scratch_shapes=[pltpu.VMEM((G, D), jnp.float32)],
    )(acc2, g2, deg2, b2r, batch3, p1, Wl1, bl1r, Wl2, bl2r)


def kernel(x, edge_index, batch, W1, b1, W2, b2, Wl1, bl1, Wl2, bl2):
    src = edge_index[0]
    dst = edge_index[1]
    zeros_nk = jnp.zeros((N, 16), jnp.float32)
    ones_ek = jnp.ones((EK, 16), jnp.float32)
    batch3 = jnp.reshape(batch, (NB, 1, BN))

    deg2 = _sc_degree(dst, zeros_nk, ones_ek)
    g1 = _tc_stage1(x, W1, deg2)
    acc1 = _sc_scatter(g1, src, dst)
    g2, p1 = _tc_stage2(acc1, g1, deg2, W2, jnp.reshape(b1, (1, D)), batch3)
    acc2 = _sc_scatter(g2, src, dst)
    h, lsm = _tc_stage3(acc2, g2, deg2, jnp.reshape(b2, (1, D)), batch3, p1,
                        Wl1, jnp.reshape(bl1, (1, 2 * D)), Wl2,
                        jnp.reshape(bl2, (1, 10)))
    return (h, lsm)


# speedup vs baseline: 11.1625x; 11.1625x over previous
import functools

import jax
import jax.numpy as jnp
from jax import lax
from jax.experimental import pallas as pl
from jax.experimental.pallas import tpu as pltpu
from jax.experimental.pallas import tpu_sc as plsc

N = 10000
NPAD = 10240
E = 320000
D = 128
G = 64
NC = 2
NS = 16
EK = 80
RPS = 640
RLAST = N - (NS - 1) * RPS
EDGES_PER_SUB = E // (NC * NS)
NCHUNK = EDGES_PER_SUB // EK
NB = 10
BN = N // NB


def _sc_mesh():
    return plsc.VectorSubcoreMesh(core_axis_name="c", subcore_axis_name="s")


def _row_slab(s, copy_fn):
    @pl.when(s < NS - 1)
    def _():
        copy_fn(s * RPS, RPS)

    @pl.when(s == NS - 1)
    def _():
        copy_fn((NS - 1) * RPS, RLAST)


def _deg_body(dst_hbm, out_hbm, didx_v, hist_v):
    c = lax.axis_index("c")
    s = lax.axis_index("s")
    zeros16 = jnp.zeros((16,), jnp.float32)

    def zstep(i, carry):
        hist_v[pl.ds(i * 16, 16)] = zeros16
        return carry

    lax.fori_loop(0, NPAD // 16, zstep, 0)
    base = c * (E // NC) + s * EDGES_PER_SUB
    pltpu.sync_copy(dst_hbm.at[pl.ds(base, EDGES_PER_SUB)], didx_v)
    ones16 = jnp.ones((16,), jnp.float32)

    def step(i, carry):
        idx = didx_v[pl.ds(i * 16, 16)]
        plsc.addupdate_scatter(hist_v, [idx], ones16)
        return carry

    lax.fori_loop(0, EDGES_PER_SUB // 16, step, 0)
    pltpu.sync_copy(hist_v, out_hbm.at[c, s])


def _sc_degree(dst):
    return pl.kernel(
        _deg_body,
        out_type=jax.ShapeDtypeStruct((NC, NS, NPAD), jnp.float32),
        mesh=_sc_mesh(),
        compiler_params=pltpu.CompilerParams(needs_layout_passes=False),
        scratch_types=[
            pltpu.VMEM((EDGES_PER_SUB,), jnp.int32),
            pltpu.VMEM((NPAD,), jnp.float32),
        ],
    )(dst)


def _scatter_body(g_hbm, src_hbm, dst_hbm, out_hbm, sidx_v, didx_v, rows_v,
                  sem, acc_sh):
    c = lax.axis_index("c")
    s = lax.axis_index("s")
    _row_slab(s, lambda r0, nr: pltpu.sync_copy(
        g_hbm.at[pl.ds(r0, nr)], acc_sh.at[pl.ds(r0, nr)]))
    plsc.subcore_barrier()
    base = c * (E // NC) + s * EDGES_PER_SUB

    def step(i, carry):
        off = base + i * EK
        pltpu.sync_copy(src_hbm.at[pl.ds(off, EK)], sidx_v)
        pltpu.sync_copy(dst_hbm.at[pl.ds(off, EK)], didx_v)
        pltpu.async_copy(g_hbm.at[sidx_v], rows_v, sem).wait()
        pltpu.sync_copy(rows_v, acc_sh.at[didx_v], add=True)
        return carry

    lax.fori_loop(0, NCHUNK, step, 0)
    plsc.subcore_barrier()
    _row_slab(s, lambda r0, nr: pltpu.sync_copy(
        acc_sh.at[pl.ds(r0, nr)], out_hbm.at[c, pl.ds(r0, nr)]))


def _sc_scatter(g, src, dst):
    return pl.kernel(
        _scatter_body,
        out_type=jax.ShapeDtypeStruct((NC, N, D), jnp.float32),
        mesh=_sc_mesh(),
        scratch_types=[
            pltpu.VMEM((EK,), jnp.int32),
            pltpu.VMEM((EK,), jnp.int32),
            pltpu.VMEM((EK, D), jnp.float32),
            pltpu.SemaphoreType.DMA,
            pltpu.VMEM_SHARED((N, D), jnp.float32),
        ],
    )(g, src, dst)


def _tc1_body(x_ref, w_ref, deg_ref, g_ref):
    deg = jnp.sum(deg_ref[...], axis=0) + 1.0
    dinv = lax.rsqrt(deg)
    h = jnp.dot(x_ref[...], w_ref[...], preferred_element_type=jnp.float32)
    g_ref[...] = h * dinv


def _tc_stage1(x, W1, deg2):
    return pl.pallas_call(
        _tc1_body,
        grid=(NB,),
        in_specs=[
            pl.BlockSpec((BN, D), lambda i: (i, 0)),
            pl.BlockSpec((D, D), lambda i: (0, 0)),
            pl.BlockSpec((NC * NS, BN, 1), lambda i: (0, i, 0)),
        ],
        out_specs=pl.BlockSpec((BN, D), lambda i: (i, 0)),
        out_shape=jax.ShapeDtypeStruct((N, D), jnp.float32),
    )(x, W1, deg2)


def _tc2_body(acc_ref, g1_ref, deg_ref, w2_ref, b1_ref, batch_ref,
              g2_ref, p1_ref, p1_acc):
    i = pl.program_id(0)
    deg = jnp.sum(deg_ref[...], axis=0) + 1.0
    dinv = lax.rsqrt(deg)
    esum = acc_ref[0] + acc_ref[1] - g1_ref[...]
    out1 = jax.nn.relu(esum * dinv + b1_ref[...])
    h2 = jnp.dot(out1, w2_ref[...], preferred_element_type=jnp.float32)
    g2_ref[...] = h2 * dinv
    onehot = (batch_ref[0] == lax.broadcasted_iota(jnp.int32, (G, BN), 0)
              ).astype(jnp.float32)
    part = jnp.dot(onehot, out1, preferred_element_type=jnp.float32)

    @pl.when(i == 0)
    def _():
        p1_acc[...] = jnp.zeros_like(p1_acc)

    p1_acc[...] += part

    @pl.when(i == NB - 1)
    def _():
        p1_ref[...] = p1_acc[...]


def _tc_stage2(acc1, g1, deg2, W2, b1r, batch3):
    return pl.pallas_call(
        _tc2_body,
        grid=(NB,),
        in_specs=[
            pl.BlockSpec((NC, BN, D), lambda i: (0, i, 0)),
            pl.BlockSpec((BN, D), lambda i: (i, 0)),
            pl.BlockSpec((NC * NS, BN, 1), lambda i: (0, i, 0)),
            pl.BlockSpec((D, D), lambda i: (0, 0)),
            pl.BlockSpec((1, D), lambda i: (0, 0)),
            pl.BlockSpec((1, 1, BN), lambda i: (i, 0, 0)),
        ],
        out_specs=[
            pl.BlockSpec((BN, D), lambda i: (i, 0)),
            pl.BlockSpec((G, D), lambda i: (0, 0)),
        ],
        out_shape=[
            jax.ShapeDtypeStruct((N, D), jnp.float32),
            jax.ShapeDtypeStruct((G, D), jnp.float32),
        ],
        scratch_shapes=[pltpu.VMEM((G, D), jnp.float32)],
    )(acc1, g1, deg2, W2, b1r, batch3)


def _tc3_body(acc_ref, g2_ref, deg_ref, b2_ref, batch_ref, p1_ref,
              wl1_ref, bl1_ref, wl2_ref, bl2_ref, h_ref, lsm_ref, p2_acc):
    i = pl.program_id(0)
    deg = jnp.sum(deg_ref[...], axis=0) + 1.0
    dinv = lax.rsqrt(deg)
    esum = acc_ref[0] + acc_ref[1] - g2_ref[...]
    out2 = jax.nn.relu(esum * dinv + b2_ref[...])
    onehot = (batch_ref[0] == lax.broadcasted_iota(jnp.int32, (G, BN), 0)
              ).astype(jnp.float32)
    part = jnp.dot(onehot, out2, preferred_element_type=jnp.float32)

    @pl.when(i == 0)
    def _():
        p2_acc[...] = jnp.zeros_like(p2_acc)

    p2_acc[...] += part

    @pl.when(i == NB - 1)
    def _():
        p = jnp.concatenate([p1_ref[...], p2_acc[...]], axis=1)
        h = jnp.dot(p, wl1_ref[...], preferred_element_type=jnp.float32)
        h = jax.nn.relu(h + bl1_ref[...])
        h = jnp.dot(h, wl2_ref[...], preferred_element_type=jnp.float32)
        h = h + bl2_ref[...]
        m = jnp.max(h, axis=1, keepdims=True)
        lse = jnp.log(jnp.sum(jnp.exp(h - m), axis=1, keepdims=True))
        h_ref[...] = h
        lsm_ref[...] = h - m - lse


def _tc_stage3(acc2, g2, deg2, b2r, batch3, p1, Wl1, bl1r, Wl2, bl2r):
    return pl.pallas_call(
        _tc3_body,
        grid=(NB,),
        in_specs=[
            pl.BlockSpec((NC, BN, D), lambda i: (0, i, 0)),
            pl.BlockSpec((BN, D), lambda i: (i, 0)),
            pl.BlockSpec((NC * NS, BN, 1), lambda i: (0, i, 0)),
            pl.BlockSpec((1, D), lambda i: (0, 0)),
            pl.BlockSpec((1, 1, BN), lambda i: (i, 0, 0)),
            pl.BlockSpec((G, D), lambda i: (0, 0)),
            pl.BlockSpec((2 * D, 2 * D), lambda i: (0, 0)),
            pl.BlockSpec((1, 2 * D), lambda i: (0, 0)),
            pl.BlockSpec((2 * D, 10), lambda i: (0, 0)),
            pl.BlockSpec((1, 10), lambda i: (0, 0)),
        ],
        out_specs=[
            pl.BlockSpec((G, 10), lambda i: (0, 0)),
            pl.BlockSpec((G, 10), lambda i: (0, 0)),
        ],
        out_shape=[
            jax.ShapeDtypeStruct((G, 10), jnp.float32),
            jax.ShapeDtypeStruct((G, 10), jnp.float32),
        ],
        scratch_shapes=[pltpu.VMEM((G, D), jnp.float32)],
    )(acc2, g2, deg2, b2r, batch3, p1, Wl1, bl1r, Wl2, bl2r)


def kernel(x, edge_index, batch, W1, b1, W2, b2, Wl1, bl1, Wl2, bl2):
    src = edge_index[0]
    dst = edge_index[1]
    batch3 = jnp.reshape(batch, (NB, 1, BN))

    deg2 = jnp.reshape(_sc_degree(dst), (NC * NS, NPAD, 1))
    g1 = _tc_stage1(x, W1, deg2)
    acc1 = _sc_scatter(g1, src, dst)
    g2, p1 = _tc_stage2(acc1, g1, deg2, W2, jnp.reshape(b1, (1, D)), batch3)
    acc2 = _sc_scatter(g2, src, dst)
    h, lsm = _tc_stage3(acc2, g2, deg2, jnp.reshape(b2, (1, D)), batch3, p1,
                        Wl1, jnp.reshape(bl1, (1, 2 * D)), Wl2,
                        jnp.reshape(bl2, (1, 10)))
    return (h, lsm)
